# Initial kernel scaffold; baseline (speedup 1.0000x reference)
import jax
import jax.numpy as jnp
from jax.experimental import pallas as pl


def kernel(pred_poses):
    raise NotImplementedError("write your pallas kernel here")


# speedup vs baseline: 17.0489x; 17.0489x over previous
import jax
import jax.numpy as jnp
from jax.experimental import pallas as pl

K = 5
LAMBDA_REP = 1.0
DELTA = 0.01
S = 2.0


def _repulsion_kernel(x_ref, out_ref):
    b = pl.program_id(0)

    @pl.when(b == 0)
    def _init():
        out_ref[...] = jnp.zeros_like(out_ref)

    x = x_ref[0]
    n = x.shape[0]
    sq = jnp.sum(x * x, axis=1)
    gram = jax.lax.dot_general(
        x, x, (((1,), (1,)), ((), ())),
        preferred_element_type=jnp.float32,
        precision=jax.lax.Precision.HIGHEST,
    )
    d2 = sq[:, None] + sq[None, :] - 2.0 * gram
    d2 = jnp.maximum(d2, 0.0)
    col = jax.lax.broadcasted_iota(jnp.int32, (n, n), 1)
    row = jax.lax.broadcasted_iota(jnp.int32, (n, n), 0)
    v = jnp.where(row == col, 0.0, LAMBDA_REP / (DELTA + d2))

    acc = jnp.zeros((n, 1), dtype=jnp.float32)
    for _ in range(K):
        m = jnp.max(v, axis=1, keepdims=True)
        acc = acc + m
        jstar = jnp.min(jnp.where(v == m, col, n), axis=1, keepdims=True)
        v = jnp.where(col == jstar, 0.0, v)

    out_ref[...] += jnp.sum(acc).reshape(1, 1)


def kernel(pred_poses):
    B, N, D = pred_poses.shape
    k_actual = min(K, N - 1)
    total = pl.pallas_call(
        _repulsion_kernel,
        grid=(B,),
        in_specs=[pl.BlockSpec((1, N, D), lambda b: (b, 0, 0))],
        out_specs=pl.BlockSpec((1, 1), lambda b: (0, 0)),
        out_shape=jax.ShapeDtypeStruct((1, 1), jnp.float32),
    )(pred_poses)
    return total[0, 0] / (B * N * k_actual)
